# baseline (speedup 1.0000x reference)
import functools

import jax
import jax.numpy as jnp
from jax import lax
from jax.experimental import pallas as pl
from jax.experimental.pallas import tpu as pltpu
from jax.experimental.pallas import tpu_sc as plsc

_N = 10000
_E = 320000
_H = 8
_D = 16
_HID = 128
_NG = 128

_NPAD = 10240
_CHUNK = 48
_TILES = 32
_CPT = 210
_EPAD = _TILES * _CPT * _CHUNK
_NTOT = _NPAD + _NPAD // 8
_APT = _NTOT // 16


def _embed_body(x_ref, w_ref, b_ref, h_ref):
    y = jnp.maximum(
        jnp.dot(x_ref[...], w_ref[...], preferred_element_type=jnp.float32)
        + b_ref[...], 0.0)
    h_ref[...] = jnp.concatenate(
        [y, jnp.zeros((_NPAD, _HID - 64), jnp.float32)], axis=1)


def _group_matrix(rows, cols):
    r = lax.broadcasted_iota(jnp.int32, (rows, cols), 0) // _D
    c = lax.broadcasted_iota(jnp.int32, (rows, cols), 1)
    return (r == c).astype(jnp.float32)


def _group_matrix_t(rows, cols):
    r = lax.broadcasted_iota(jnp.int32, (rows, cols), 0)
    c = lax.broadcasted_iota(jnp.int32, (rows, cols), 1) // _D
    return (r == c).astype(jnp.float32)


def _prep_body(h_ref, wl_ref, bl_ref, wr_ref, br_ref, ew_ref, att_ref,
               xl_ref, xr_ref, c_ref):
    hb = h_ref[...]
    xl = jnp.dot(hb, wl_ref[...], preferred_element_type=jnp.float32) + bl_ref[...]
    xr = jnp.dot(hb, wr_ref[...], preferred_element_type=jnp.float32) + br_ref[...]
    xl_ref[...] = xl
    xr_ref[...] = xr
    s = xl + xr + ew_ref[...]
    s = jnp.maximum(s, 0.2 * s) * att_ref[...]
    c_ref[...] = jnp.exp(jnp.dot(s, _group_matrix(_HID, _HID),
                                 preferred_element_type=jnp.float32))


def _combine_body(num_ref, den_ref, xl_ref, es_ref, bias_ref, out_ref,
                  s1_ref, s2_ref):
    es_b = jnp.dot(es_ref[...], _group_matrix_t(_HID, _HID),
                   preferred_element_type=jnp.float32)
    num = num_ref[0] + num_ref[1] + es_b * xl_ref[...]
    den = den_ref[0] + den_ref[1]
    den_b = jnp.dot(den, _group_matrix_t(_D, _HID),
                    preferred_element_type=jnp.float32) + es_b
    out = num / den_b + bias_ref[...]
    mask = lax.broadcasted_iota(jnp.int32, (_NPAD, _HID), 0) < _N
    out = jnp.where(mask, out, 0.0)
    out_ref[...] = out
    s1_ref[...] = jnp.sum(out, axis=0, keepdims=True)
    s2_ref[...] = jnp.sum(out * out, axis=0, keepdims=True)


def _norm_body(out_ref, s1_ref, s2_ref, g_ref, b_ref, alpha_ref, res_ref,
               h_ref):
    mu = s1_ref[...] / float(_N)
    var = s2_ref[...] / float(_N) - mu * mu
    inv = lax.rsqrt(var + 1e-5)
    y = (out_ref[...] - mu) * inv * g_ref[...] + b_ref[...]
    y = y + jnp.broadcast_to(alpha_ref[...], (_NPAD, _HID)) * res_ref[...]
    mask = lax.broadcasted_iota(jnp.int32, (_NPAD, _HID), 0) < _N
    h_ref[...] = jnp.where(mask, jnp.maximum(y, 0.0), 0.0)


def _pool_body(h_ref, batch_ref, sums_ref, cnt_ref):
    b = batch_ref[...]
    oh = (jnp.broadcast_to(b, (_NG, _NPAD))
          == lax.broadcasted_iota(jnp.int32, (_NG, _NPAD), 0)).astype(jnp.float32)
    sums_ref[...] = jnp.dot(oh, h_ref[...], preferred_element_type=jnp.float32)
    cnt_ref[...] = jnp.dot(oh, jnp.ones((_NPAD, _HID), jnp.float32),
                           preferred_element_type=jnp.float32)


def _head_body(sums_ref, cnt_ref, meta_in_ref, graph_in_ref,
               w1m, b1m, w2m, b2m, w1g, b1g, w2g, b2g,
               f1w, f1b, f2w, f2b, f3w, f3b, out_ref):
    pooled = sums_ref[...] / jnp.maximum(cnt_ref[...], 1.0)
    meta = jnp.maximum(meta_in_ref[...] @ w1m[...] + b1m[...], 0.0)
    meta = jnp.maximum(meta @ w2m[...] + b2m[...], 0.0)
    graph = jnp.maximum(graph_in_ref[...] @ w1g[...] + b1g[...], 0.0)
    graph = jnp.maximum(graph @ w2g[...] + b2g[...], 0.0)
    z = jnp.concatenate([pooled, meta, graph], axis=1)
    z = jnp.maximum(z @ f1w[...] + f1b[...], 0.0)
    z = jnp.maximum(z @ f2w[...] + f2b[...], 0.0)
    out_ref[...] = z @ f3w[...] + f3b[...]


def _tc_call(body, out_shapes, *args):
    return pl.pallas_call(
        body,
        out_shape=out_shapes,
    )(*args)


def _edge_body(src_hbm, dst_hbm, ea_hbm, xl_hbm, xr_hbm, w_hbm,
               num_out,
               idx_s, idx_d, idx_dn, ea_v, xl_v, xr_v, den_v, w_v,
               acc_num, sem_i, sem_g, sem_sn, sem_sd):
    core = lax.axis_index("c")
    sub = lax.axis_index("s")
    wid = core * 16 + sub

    zeros16 = jnp.zeros((16,), jnp.float32)

    def zrow(j, _):
        for h in range(_H):
            den_v[j, pl.ds(h * _D, _D)] = zeros16
        return 0

    lax.fori_loop(0, _CHUNK, zrow, 0)
    row0 = sub * _APT
    for b in range(_APT // _CHUNK):
        pltpu.sync_copy(den_v, acc_num.at[pl.ds(row0 + b * _CHUNK, _CHUNK)])
    plsc.subcore_barrier()

    pltpu.sync_copy(w_hbm, w_v)
    lanes = lax.iota(jnp.int32, 16)

    def ebase(ch):
        return (wid * _CPT + ch) * _CHUNK

    def issue_idx(ch, b3):
        base = ebase(ch)
        pltpu.async_copy(src_hbm.at[pl.ds(base, _CHUNK)], idx_s.at[b3], sem_i)
        pltpu.async_copy(dst_hbm.at[pl.ds(base, _CHUNK)], idx_d.at[b3], sem_i)
        pltpu.async_copy(ea_hbm.at[pl.ds(base, _CHUNK)], ea_v.at[b3], sem_i)

    def wait_idx(ch, b3):
        base = ebase(ch)
        pltpu.make_async_copy(src_hbm.at[pl.ds(base, _CHUNK)],
                              idx_s.at[b3], sem_i).wait()
        pltpu.make_async_copy(dst_hbm.at[pl.ds(base, _CHUNK)],
                              idx_d.at[b3], sem_i).wait()
        pltpu.make_async_copy(ea_hbm.at[pl.ds(base, _CHUNK)],
                              ea_v.at[b3], sem_i).wait()

    def issue_gather(g2, b3):
        pltpu.async_copy(xl_hbm.at[idx_s.at[b3]], xl_v.at[g2], sem_g)
        pltpu.async_copy(xr_hbm.at[idx_d.at[b3]], xr_v.at[g2], sem_g)

    def wait_gather(g2, b3):
        pltpu.make_async_copy(xl_hbm.at[idx_s.at[b3]],
                              xl_v.at[g2], sem_g).wait()
        pltpu.make_async_copy(xr_hbm.at[idx_d.at[b3]],
                              xr_v.at[g2], sem_g).wait()

    def zgroup_for(b3):
        def zg(g, _):
            dst16 = idx_d[b3, pl.ds(g * 16, 16)]
            dbase = (dst16 & 7) * _D
            rows = g * 16 + lanes
            for h in range(_H):
                plsc.store_scatter(den_v, [rows, dbase + h], zeros16)
            return 0

        lax.fori_loop(0, _CHUNK // 16, zg, 0)

    zv = jnp.full((16,), 0, jnp.int32)
    ov = jnp.full((16,), 1, jnp.int32)

    def compute(g2, b3):
        xlb, xrb = xl_v.at[g2], xr_v.at[g2]

        def group(g, _):
            rows = g * 16 + lanes
            ea16 = ea_v[b3, pl.ds(g * 16, 16)]
            dst16 = idx_d[b3, pl.ds(g * 16, 16)]
            idx_dn[pl.ds(g * 16, 16)] = _NPAD + (dst16 >> 3)
            dbase = (dst16 & 7) * _D

            def hbody(h, _):
                colbase = jnp.full((16,), h * _D, jnp.int32)
                acc = None
                xls = []
                for d in range(_D):
                    rotv = (lanes + d) & (_D - 1)
                    colv = colbase + rotv
                    xlv = plsc.load_gather(xlb, [rows, colv])
                    xrv = plsc.load_gather(xrb, [rows, colv])
                    wv = plsc.load_gather(w_v, [zv, colv])
                    av = plsc.load_gather(w_v, [ov, colv])
                    xls.append(xlv)
                    s = xlv + xrv + ea16 * wv
                    s = jnp.maximum(s, 0.2 * s) * av
                    acc = s if acc is None else acc + s
                exv = jnp.exp(acc)
                for d in range(_D):
                    rotv = (lanes + d) & (_D - 1)
                    plsc.store_scatter(xrb, [rows, colbase + rotv],
                                       exv * xls[d])
                plsc.store_scatter(den_v, [rows, dbase + h], exv)
                return 0

            lax.fori_loop(0, _H, hbody, 0)
            return 0

        lax.fori_loop(0, _CHUNK // 16, group, 0)

    def wait_sn(g2, b3):
        pltpu.make_async_copy(xr_v.at[g2],
                              acc_num.at[idx_d.at[b3]], sem_sn).wait()

    def wait_sd():
        pltpu.make_async_copy(den_v, acc_num.at[idx_dn], sem_sd).wait()

    issue_idx(0, 0)
    wait_idx(0, 0)
    issue_gather(0, 0)
    issue_idx(1, 1)

    def chunk_six(c6, _):
        for par in range(6):
            ch = c6 * 6 + par
            g2, b3 = par % 2, par % 3
            wait_gather(g2, b3)

            @pl.when(ch > 0)
            def _():
                wait_sn(1 - g2, (par - 1) % 3)

            @pl.when(ch + 1 < _CPT)
            def _():
                wait_idx(ch + 1, (par + 1) % 3)
                issue_gather(1 - g2, (par + 1) % 3)

            compute(g2, b3)
            pltpu.sync_copy(den_v, acc_num.at[idx_dn], add=True)
            zgroup_for(b3)
            pltpu.async_copy(xr_v.at[g2], acc_num.at[idx_d.at[b3]],
                             sem_sn, add=True)

            @pl.when(ch + 2 < _CPT)
            def _():
                issue_idx(ch + 2, (par + 2) % 3)
        return 0

    lax.fori_loop(0, _CPT // 6, chunk_six, 0)
    wait_sn(1, 2)
    plsc.subcore_barrier()

    for b in range(_APT // _CHUNK):
        r0 = row0 + b * _CHUNK
        pltpu.sync_copy(acc_num.at[pl.ds(r0, _CHUNK)], xl_v.at[0])
        pltpu.sync_copy(xl_v.at[0], num_out.at[core, pl.ds(r0, _CHUNK)])


@functools.cache
def _edge_pass():
  return pl.kernel(
    _edge_body,
    out_type=jax.ShapeDtypeStruct((2, _NTOT, _HID), jnp.float32),
    mesh=plsc.VectorSubcoreMesh(core_axis_name="c", subcore_axis_name="s"),
    compiler_params=pltpu.CompilerParams(needs_layout_passes=False),
    scratch_types=[
        pltpu.VMEM((3, _CHUNK), jnp.int32),
        pltpu.VMEM((3, _CHUNK), jnp.int32),
        pltpu.VMEM((_CHUNK,), jnp.int32),
        pltpu.VMEM((3, _CHUNK), jnp.float32),
        pltpu.VMEM((2, _CHUNK, _HID), jnp.float32),
        pltpu.VMEM((2, _CHUNK, _HID), jnp.float32),
        pltpu.VMEM((_CHUNK, _HID), jnp.float32),
        pltpu.VMEM((2, _HID), jnp.float32),
        pltpu.VMEM_SHARED((_NTOT, _HID), jnp.float32),
        pltpu.SemaphoreType.DMA,
        pltpu.SemaphoreType.DMA,
        pltpu.SemaphoreType.DMA,
        pltpu.SemaphoreType.DMA,
    ],
  )


def _layer(h, src_p, dst_p, ea_p, xs):
    Wl, bl, Wr, br, ew, att, wtab, bias, g, b, alpha = xs
    xl, xr, c = _tc_call(
        _prep_body,
        [jax.ShapeDtypeStruct((_NPAD, _HID), jnp.float32),
         jax.ShapeDtypeStruct((_NPAD, _HID), jnp.float32),
         jax.ShapeDtypeStruct((_NPAD, _HID), jnp.float32)],
        h, Wl, bl, Wr, br, ew, att)
    acc2 = _edge_pass()(src_p, dst_p, ea_p, xl, xr, wtab)
    num2 = acc2[:, :_NPAD, :]
    den2 = acc2[:, _NPAD:, :].reshape(2, _NPAD, _D)
    out, s1, s2 = _tc_call(
        _combine_body,
        [jax.ShapeDtypeStruct((_NPAD, _HID), jnp.float32),
         jax.ShapeDtypeStruct((1, _HID), jnp.float32),
         jax.ShapeDtypeStruct((1, _HID), jnp.float32)],
        num2, den2, xl, c, bias)
    return _tc_call(
        _norm_body, jax.ShapeDtypeStruct((_NPAD, _HID), jnp.float32),
        out, s1, s2, g, b, alpha, h)


def kernel(x, edge_index, edge_attr, batch, global_features, params):
    ea_mean = edge_attr.mean()

    x_p = jnp.zeros((_NPAD, 8), jnp.float32).at[:_N, :4].set(x)
    pad_e = _EPAD - _E
    src_p = jnp.concatenate([edge_index[0], jnp.zeros((pad_e,), jnp.int32)])
    dst_p = jnp.concatenate(
        [edge_index[1], jnp.full((pad_e,), _NPAD - 1, jnp.int32)])
    ea_p = jnp.concatenate([edge_attr[:, 0], jnp.zeros((pad_e,), jnp.float32)])
    batch_p = jnp.concatenate(
        [batch, jnp.full((_NPAD - _N,), -1, jnp.int32)]).reshape(1, _NPAD)

    Wls, bls, Wrs, brs, ews, atts, wtabs, biases, gs, bs = (
        [] for _ in range(10))
    for i, name in enumerate(("gat1", "gat2", "gat3", "gat4")):
        p = params[name]
        Wl, Wr = p["Wl"], p["Wr"]
        if i == 0:
            Wl = jnp.zeros((_HID, _HID), jnp.float32).at[:64].set(Wl)
            Wr = jnp.zeros((_HID, _HID), jnp.float32).at[:64].set(Wr)
        Wls.append(Wl)
        Wrs.append(Wr)
        bls.append(p["bl"].reshape(1, _HID))
        brs.append(p["br"].reshape(1, _HID))
        ews.append((ea_mean * p["We"][0]).reshape(1, _HID))
        atts.append(p["att"].reshape(1, _HID))
        wtabs.append(jnp.stack([p["We"][0], p["att"].reshape(-1)]))
        biases.append(p["bias"].reshape(1, _HID))
        g, b = params["bn" + str(i + 1)]
        gs.append(g.reshape(1, _HID))
        bs.append(b.reshape(1, _HID))
    xs = tuple(jnp.stack(v) for v in
               (Wls, bls, Wrs, brs, ews, atts, wtabs, biases, gs, bs))
    xs = xs + (jnp.array([0.0, 1.0, 1.0, 1.0],
                         jnp.float32).reshape(4, 1, 1),)

    We_, be_ = params["embed"]
    We_p = jnp.zeros((8, 64), jnp.float32).at[:4].set(We_)

    h = _tc_call(_embed_body,
                 jax.ShapeDtypeStruct((_NPAD, _HID), jnp.float32),
                 x_p, We_p, be_.reshape(1, -1))

    def body(hc, x):
        return _layer(hc, src_p, dst_p, ea_p, x), None

    h, _ = lax.scan(body, h, xs)

    sums, cnt = _tc_call(
        _pool_body,
        [jax.ShapeDtypeStruct((_NG, _HID), jnp.float32),
         jax.ShapeDtypeStruct((_NG, _HID), jnp.float32)],
        h, batch_p)

    gf = global_features.squeeze(1)
    p = params
    return _tc_call(
        _head_body, jax.ShapeDtypeStruct((_NG, 1), jnp.float32),
        sums, cnt, gf[:, 0:4], gf[:, 4:6],
        p["meta1"][0], p["meta1"][1], p["meta2"][0], p["meta2"][1],
        p["graph1"][0], p["graph1"][1], p["graph2"][0], p["graph2"][1],
        p["fc1"][0], p["fc1"][1], p["fc2"][0], p["fc2"][1],
        p["fc3"][0], p["fc3"][1])


# speedup vs baseline: 1.0280x; 1.0280x over previous
import functools

import jax
import jax.numpy as jnp
from jax import lax
from jax.experimental import pallas as pl
from jax.experimental.pallas import tpu as pltpu
from jax.experimental.pallas import tpu_sc as plsc

_N = 10000
_E = 320000
_H = 8
_D = 16
_HID = 128
_NG = 128

_NPAD = 10240
_CHUNK = 48
_TILES = 32
_CPT = 210
_EPAD = _TILES * _CPT * _CHUNK
_NTOT = _NPAD + _NPAD // 8
_APT = _NTOT // 16


def _embed_body(x_ref, w_ref, b_ref, h_ref):
    y = jnp.maximum(
        jnp.dot(x_ref[...], w_ref[...], preferred_element_type=jnp.float32)
        + b_ref[...], 0.0)
    h_ref[...] = jnp.concatenate(
        [y, jnp.zeros((_NPAD, _HID - 64), jnp.float32)], axis=1)


def _group_matrix(rows, cols):
    r = lax.broadcasted_iota(jnp.int32, (rows, cols), 0) // _D
    c = lax.broadcasted_iota(jnp.int32, (rows, cols), 1)
    return (r == c).astype(jnp.float32)


def _group_matrix_t(rows, cols):
    r = lax.broadcasted_iota(jnp.int32, (rows, cols), 0)
    c = lax.broadcasted_iota(jnp.int32, (rows, cols), 1) // _D
    return (r == c).astype(jnp.float32)


def _prep_body(h_ref, wl_ref, bl_ref, wr_ref, br_ref, ew_ref, att_ref,
               xl_ref, xr_ref, c_ref):
    hb = h_ref[...]
    xl = jnp.dot(hb, wl_ref[...], preferred_element_type=jnp.float32) + bl_ref[...]
    xr = jnp.dot(hb, wr_ref[...], preferred_element_type=jnp.float32) + br_ref[...]
    xl_ref[...] = xl
    xr_ref[...] = xr
    s = xl + xr + ew_ref[...]
    s = jnp.maximum(s, 0.2 * s) * att_ref[...]
    c_ref[...] = jnp.exp(jnp.dot(s, _group_matrix(_HID, _HID),
                                 preferred_element_type=jnp.float32))


def _combine_body(num_ref, den_ref, xl_ref, es_ref, bias_ref, out_ref,
                  s1_ref, s2_ref):
    es_b = jnp.dot(es_ref[...], _group_matrix_t(_HID, _HID),
                   preferred_element_type=jnp.float32)
    num = num_ref[0] + num_ref[1] + es_b * xl_ref[...]
    den = den_ref[0] + den_ref[1]
    den_b = jnp.dot(den, _group_matrix_t(_D, _HID),
                    preferred_element_type=jnp.float32) + es_b
    out = num / den_b + bias_ref[...]
    mask = lax.broadcasted_iota(jnp.int32, (_NPAD, _HID), 0) < _N
    out = jnp.where(mask, out, 0.0)
    out_ref[...] = out
    s1_ref[...] = jnp.sum(out, axis=0, keepdims=True)
    s2_ref[...] = jnp.sum(out * out, axis=0, keepdims=True)


def _norm_body(out_ref, s1_ref, s2_ref, g_ref, b_ref, alpha_ref, res_ref,
               h_ref):
    mu = s1_ref[...] / float(_N)
    var = s2_ref[...] / float(_N) - mu * mu
    inv = lax.rsqrt(var + 1e-5)
    y = (out_ref[...] - mu) * inv * g_ref[...] + b_ref[...]
    y = y + jnp.broadcast_to(alpha_ref[...], (_NPAD, _HID)) * res_ref[...]
    mask = lax.broadcasted_iota(jnp.int32, (_NPAD, _HID), 0) < _N
    h_ref[...] = jnp.where(mask, jnp.maximum(y, 0.0), 0.0)


def _pool_body(h_ref, batch_ref, sums_ref, cnt_ref):
    b = batch_ref[...]
    oh = (jnp.broadcast_to(b, (_NG, _NPAD))
          == lax.broadcasted_iota(jnp.int32, (_NG, _NPAD), 0)).astype(jnp.float32)
    sums_ref[...] = jnp.dot(oh, h_ref[...], preferred_element_type=jnp.float32)
    cnt_ref[...] = jnp.dot(oh, jnp.ones((_NPAD, _HID), jnp.float32),
                           preferred_element_type=jnp.float32)


def _head_body(sums_ref, cnt_ref, meta_in_ref, graph_in_ref,
               w1m, b1m, w2m, b2m, w1g, b1g, w2g, b2g,
               f1w, f1b, f2w, f2b, f3w, f3b, out_ref):
    pooled = sums_ref[...] / jnp.maximum(cnt_ref[...], 1.0)
    meta = jnp.maximum(meta_in_ref[...] @ w1m[...] + b1m[...], 0.0)
    meta = jnp.maximum(meta @ w2m[...] + b2m[...], 0.0)
    graph = jnp.maximum(graph_in_ref[...] @ w1g[...] + b1g[...], 0.0)
    graph = jnp.maximum(graph @ w2g[...] + b2g[...], 0.0)
    z = jnp.concatenate([pooled, meta, graph], axis=1)
    z = jnp.maximum(z @ f1w[...] + f1b[...], 0.0)
    z = jnp.maximum(z @ f2w[...] + f2b[...], 0.0)
    out_ref[...] = z @ f3w[...] + f3b[...]


def _tc_call(body, out_shapes, *args):
    return pl.pallas_call(
        body,
        out_shape=out_shapes,
    )(*args)


def _edge_body(src_hbm, dst_hbm, ea_hbm, xl_hbm, xr_hbm, w_hbm,
               num_out,
               idx_s, idx_d, idx_dn, ea_v, xl_v, xr_v, den_v, w_v,
               acc_num, sem_i, sem_g, sem_sn, sem_sd):
    core = lax.axis_index("c")
    sub = lax.axis_index("s")
    wid = core * 16 + sub

    zeros16 = jnp.zeros((16,), jnp.float32)

    def zrow(j, _):
        for h in range(_H):
            den_v[j, pl.ds(h * _D, _D)] = zeros16
        return 0

    lax.fori_loop(0, _CHUNK, zrow, 0)
    row0 = sub * _APT
    for b in range(_APT // _CHUNK):
        pltpu.sync_copy(den_v, acc_num.at[pl.ds(row0 + b * _CHUNK, _CHUNK)])
    plsc.subcore_barrier()

    pltpu.sync_copy(w_hbm, w_v)
    lanes = lax.iota(jnp.int32, 16)

    def ebase(ch):
        return (wid * _CPT + ch) * _CHUNK

    def issue_idx(ch, b3):
        base = ebase(ch)
        pltpu.async_copy(src_hbm.at[pl.ds(base, _CHUNK)], idx_s.at[b3], sem_i)
        pltpu.async_copy(dst_hbm.at[pl.ds(base, _CHUNK)], idx_d.at[b3], sem_i)
        pltpu.async_copy(ea_hbm.at[pl.ds(base, _CHUNK)], ea_v.at[b3], sem_i)

    def wait_idx(ch, b3):
        base = ebase(ch)
        pltpu.make_async_copy(src_hbm.at[pl.ds(base, _CHUNK)],
                              idx_s.at[b3], sem_i).wait()
        pltpu.make_async_copy(dst_hbm.at[pl.ds(base, _CHUNK)],
                              idx_d.at[b3], sem_i).wait()
        pltpu.make_async_copy(ea_hbm.at[pl.ds(base, _CHUNK)],
                              ea_v.at[b3], sem_i).wait()

    def issue_gather(g2, b3):
        pltpu.async_copy(xl_hbm.at[idx_s.at[b3]], xl_v.at[g2], sem_g)
        pltpu.async_copy(xr_hbm.at[idx_d.at[b3]], xr_v.at[g2], sem_g)

    def wait_gather(g2, b3):
        pltpu.make_async_copy(xl_hbm.at[idx_s.at[b3]],
                              xl_v.at[g2], sem_g).wait()
        pltpu.make_async_copy(xr_hbm.at[idx_d.at[b3]],
                              xr_v.at[g2], sem_g).wait()

    def zgroup_for(b3):
        def zg(g, _):
            dst16 = idx_d[b3, pl.ds(g * 16, 16)]
            dbase = (dst16 & 7) * _D
            rows = g * 16 + lanes
            for h in range(_H):
                plsc.store_scatter(den_v, [rows, dbase + h], zeros16)
            return 0

        lax.fori_loop(0, _CHUNK // 16, zg, 0)

    zv = jnp.full((16,), 0, jnp.int32)
    ov = jnp.full((16,), 1, jnp.int32)

    def compute(g2, b3):
        xlb, xrb = xl_v.at[g2], xr_v.at[g2]

        def group(g, _):
            rows = g * 16 + lanes
            ea16 = ea_v[b3, pl.ds(g * 16, 16)]
            dst16 = idx_d[b3, pl.ds(g * 16, 16)]
            idx_dn[pl.ds(g * 16, 16)] = _NPAD + (dst16 >> 3)
            dbase = (dst16 & 7) * _D

            def hbody(h, _):
                colbase = jnp.full((16,), h * _D, jnp.int32)
                acc = None
                xls = []
                for d in range(_D):
                    rotv = (lanes + d) & (_D - 1)
                    colv = colbase + rotv
                    xlv = plsc.load_gather(xlb, [rows, colv])
                    xrv = plsc.load_gather(xrb, [rows, colv])
                    wv = plsc.load_gather(w_v, [zv, colv])
                    av = plsc.load_gather(w_v, [ov, colv])
                    xls.append(xlv)
                    s = xlv + xrv + ea16 * wv
                    s = jnp.maximum(s, 0.2 * s) * av
                    acc = s if acc is None else acc + s
                exv = jnp.exp(acc)
                for d in range(_D):
                    rotv = (lanes + d) & (_D - 1)
                    plsc.store_scatter(xrb, [rows, colbase + rotv],
                                       exv * xls[d])
                plsc.store_scatter(den_v, [rows, dbase + h], exv)
                return 0

            lax.fori_loop(0, _H, hbody, 0)
            return 0

        lax.fori_loop(0, _CHUNK // 16, group, 0)

    def wait_sn(g2, b3):
        pltpu.make_async_copy(xr_v.at[g2],
                              acc_num.at[idx_d.at[b3]], sem_sn).wait()

    def wait_sd():
        pltpu.make_async_copy(den_v, acc_num.at[idx_dn], sem_sd).wait()

    issue_idx(0, 0)
    wait_idx(0, 0)
    issue_gather(0, 0)
    issue_idx(1, 1)

    def chunk_six(c6, _):
        for par in range(6):
            ch = c6 * 6 + par
            g2, b3 = par % 2, par % 3
            wait_gather(g2, b3)

            @pl.when(ch > 0)
            def _():
                wait_sn(1 - g2, (par - 1) % 3)

            @pl.when(ch + 1 < _CPT)
            def _():
                wait_idx(ch + 1, (par + 1) % 3)
                issue_gather(1 - g2, (par + 1) % 3)

            @pl.when(ch + 2 < _CPT)
            def _():
                issue_idx(ch + 2, (par + 2) % 3)

            compute(g2, b3)
            pltpu.sync_copy(den_v, acc_num.at[idx_dn], add=True)
            zgroup_for(b3)
            pltpu.async_copy(xr_v.at[g2], acc_num.at[idx_d.at[b3]],
                             sem_sn, add=True)
        return 0

    lax.fori_loop(0, _CPT // 6, chunk_six, 0)
    wait_sn(1, 2)
    plsc.subcore_barrier()

    for b in range(_APT // _CHUNK):
        r0 = row0 + b * _CHUNK
        pltpu.sync_copy(acc_num.at[pl.ds(r0, _CHUNK)], xl_v.at[0])
        pltpu.sync_copy(xl_v.at[0], num_out.at[core, pl.ds(r0, _CHUNK)])


@functools.cache
def _edge_pass():
  return pl.kernel(
    _edge_body,
    out_type=jax.ShapeDtypeStruct((2, _NTOT, _HID), jnp.float32),
    mesh=plsc.VectorSubcoreMesh(core_axis_name="c", subcore_axis_name="s"),
    compiler_params=pltpu.CompilerParams(needs_layout_passes=False),
    scratch_types=[
        pltpu.VMEM((3, _CHUNK), jnp.int32),
        pltpu.VMEM((3, _CHUNK), jnp.int32),
        pltpu.VMEM((_CHUNK,), jnp.int32),
        pltpu.VMEM((3, _CHUNK), jnp.float32),
        pltpu.VMEM((2, _CHUNK, _HID), jnp.float32),
        pltpu.VMEM((2, _CHUNK, _HID), jnp.float32),
        pltpu.VMEM((_CHUNK, _HID), jnp.float32),
        pltpu.VMEM((2, _HID), jnp.float32),
        pltpu.VMEM_SHARED((_NTOT, _HID), jnp.float32),
        pltpu.SemaphoreType.DMA,
        pltpu.SemaphoreType.DMA,
        pltpu.SemaphoreType.DMA,
        pltpu.SemaphoreType.DMA,
    ],
  )


def _layer(h, src_p, dst_p, ea_p, xs):
    Wl, bl, Wr, br, ew, att, wtab, bias, g, b, alpha = xs
    xl, xr, c = _tc_call(
        _prep_body,
        [jax.ShapeDtypeStruct((_NPAD, _HID), jnp.float32),
         jax.ShapeDtypeStruct((_NPAD, _HID), jnp.float32),
         jax.ShapeDtypeStruct((_NPAD, _HID), jnp.float32)],
        h, Wl, bl, Wr, br, ew, att)
    acc2 = _edge_pass()(src_p, dst_p, ea_p, xl, xr, wtab)
    num2 = acc2[:, :_NPAD, :]
    den2 = acc2[:, _NPAD:, :].reshape(2, _NPAD, _D)
    out, s1, s2 = _tc_call(
        _combine_body,
        [jax.ShapeDtypeStruct((_NPAD, _HID), jnp.float32),
         jax.ShapeDtypeStruct((1, _HID), jnp.float32),
         jax.ShapeDtypeStruct((1, _HID), jnp.float32)],
        num2, den2, xl, c, bias)
    return _tc_call(
        _norm_body, jax.ShapeDtypeStruct((_NPAD, _HID), jnp.float32),
        out, s1, s2, g, b, alpha, h)


def kernel(x, edge_index, edge_attr, batch, global_features, params):
    ea_mean = edge_attr.mean()

    x_p = jnp.zeros((_NPAD, 8), jnp.float32).at[:_N, :4].set(x)
    pad_e = _EPAD - _E
    src_p = jnp.concatenate([edge_index[0], jnp.zeros((pad_e,), jnp.int32)])
    dst_p = jnp.concatenate(
        [edge_index[1], jnp.full((pad_e,), _NPAD - 1, jnp.int32)])
    ea_p = jnp.concatenate([edge_attr[:, 0], jnp.zeros((pad_e,), jnp.float32)])
    batch_p = jnp.concatenate(
        [batch, jnp.full((_NPAD - _N,), -1, jnp.int32)]).reshape(1, _NPAD)

    Wls, bls, Wrs, brs, ews, atts, wtabs, biases, gs, bs = (
        [] for _ in range(10))
    for i, name in enumerate(("gat1", "gat2", "gat3", "gat4")):
        p = params[name]
        Wl, Wr = p["Wl"], p["Wr"]
        if i == 0:
            Wl = jnp.zeros((_HID, _HID), jnp.float32).at[:64].set(Wl)
            Wr = jnp.zeros((_HID, _HID), jnp.float32).at[:64].set(Wr)
        Wls.append(Wl)
        Wrs.append(Wr)
        bls.append(p["bl"].reshape(1, _HID))
        brs.append(p["br"].reshape(1, _HID))
        ews.append((ea_mean * p["We"][0]).reshape(1, _HID))
        atts.append(p["att"].reshape(1, _HID))
        wtabs.append(jnp.stack([p["We"][0], p["att"].reshape(-1)]))
        biases.append(p["bias"].reshape(1, _HID))
        g, b = params["bn" + str(i + 1)]
        gs.append(g.reshape(1, _HID))
        bs.append(b.reshape(1, _HID))
    xs = tuple(jnp.stack(v) for v in
               (Wls, bls, Wrs, brs, ews, atts, wtabs, biases, gs, bs))
    xs = xs + (jnp.array([0.0, 1.0, 1.0, 1.0],
                         jnp.float32).reshape(4, 1, 1),)

    We_, be_ = params["embed"]
    We_p = jnp.zeros((8, 64), jnp.float32).at[:4].set(We_)

    h = _tc_call(_embed_body,
                 jax.ShapeDtypeStruct((_NPAD, _HID), jnp.float32),
                 x_p, We_p, be_.reshape(1, -1))

    def body(hc, x):
        return _layer(hc, src_p, dst_p, ea_p, x), None

    h, _ = lax.scan(body, h, xs)

    sums, cnt = _tc_call(
        _pool_body,
        [jax.ShapeDtypeStruct((_NG, _HID), jnp.float32),
         jax.ShapeDtypeStruct((_NG, _HID), jnp.float32)],
        h, batch_p)

    gf = global_features.squeeze(1)
    p = params
    return _tc_call(
        _head_body, jax.ShapeDtypeStruct((_NG, 1), jnp.float32),
        sums, cnt, gf[:, 0:4], gf[:, 4:6],
        p["meta1"][0], p["meta1"][1], p["meta2"][0], p["meta2"][1],
        p["graph1"][0], p["graph1"][1], p["graph2"][0], p["graph2"][1],
        p["fc1"][0], p["fc1"][1], p["fc2"][0], p["fc2"][1],
        p["fc3"][0], p["fc3"][1])


# speedup vs baseline: 1.4623x; 1.4224x over previous
import functools

import jax
import jax.numpy as jnp
from jax import lax
from jax.experimental import pallas as pl
from jax.experimental.pallas import tpu as pltpu
from jax.experimental.pallas import tpu_sc as plsc

_N = 10000
_E = 320000
_H = 8
_D = 16
_HID = 128
_NG = 128

_NPAD = 10240
_CHUNK = 48
_TILES = 32
_CPT = 210
_EPAD = _TILES * _CPT * _CHUNK
_NTOT = _NPAD + _NPAD // 8
_APT = _NTOT // 16


def _embed_body(x_ref, w_ref, b_ref, h_ref):
    y = jnp.maximum(
        jnp.dot(x_ref[...], w_ref[...], preferred_element_type=jnp.float32)
        + b_ref[...], 0.0)
    h_ref[...] = jnp.concatenate(
        [y, jnp.zeros((_NPAD, _HID - 64), jnp.float32)], axis=1)


def _group_matrix(rows, cols):
    r = lax.broadcasted_iota(jnp.int32, (rows, cols), 0) // _D
    c = lax.broadcasted_iota(jnp.int32, (rows, cols), 1)
    return (r == c).astype(jnp.float32)


def _group_matrix_t(rows, cols):
    r = lax.broadcasted_iota(jnp.int32, (rows, cols), 0)
    c = lax.broadcasted_iota(jnp.int32, (rows, cols), 1) // _D
    return (r == c).astype(jnp.float32)


def _prep_body(h_ref, wl_ref, bl_ref, wr_ref, br_ref, ew_ref, att_ref,
               xl_ref, xr_ref, c_ref):
    hb = h_ref[...]
    xl = jnp.dot(hb, wl_ref[...], preferred_element_type=jnp.float32) + bl_ref[...]
    xr = jnp.dot(hb, wr_ref[...], preferred_element_type=jnp.float32) + br_ref[...]
    xl_ref[...] = xl
    xr_ref[...] = xr
    s = xl + xr + ew_ref[...]
    s = jnp.maximum(s, 0.2 * s) * att_ref[...]
    c_ref[...] = jnp.exp(jnp.dot(s, _group_matrix(_HID, _HID),
                                 preferred_element_type=jnp.float32))


def _combine_body(num_ref, den_ref, xl_ref, es_ref, bias_ref, out_ref,
                  s1_ref, s2_ref):
    es_b = jnp.dot(es_ref[...], _group_matrix_t(_HID, _HID),
                   preferred_element_type=jnp.float32)
    num = num_ref[0] + num_ref[1] + es_b * xl_ref[...]
    den = den_ref[0] + den_ref[1]
    den_b = jnp.dot(den, _group_matrix_t(_D, _HID),
                    preferred_element_type=jnp.float32) + es_b
    out = num / den_b + bias_ref[...]
    mask = lax.broadcasted_iota(jnp.int32, (_NPAD, _HID), 0) < _N
    out = jnp.where(mask, out, 0.0)
    out_ref[...] = out
    s1_ref[...] = jnp.sum(out, axis=0, keepdims=True)
    s2_ref[...] = jnp.sum(out * out, axis=0, keepdims=True)


def _norm_body(out_ref, s1_ref, s2_ref, g_ref, b_ref, alpha_ref, res_ref,
               h_ref):
    mu = s1_ref[...] / float(_N)
    var = s2_ref[...] / float(_N) - mu * mu
    inv = lax.rsqrt(var + 1e-5)
    y = (out_ref[...] - mu) * inv * g_ref[...] + b_ref[...]
    y = y + jnp.broadcast_to(alpha_ref[...], (_NPAD, _HID)) * res_ref[...]
    mask = lax.broadcasted_iota(jnp.int32, (_NPAD, _HID), 0) < _N
    h_ref[...] = jnp.where(mask, jnp.maximum(y, 0.0), 0.0)


def _pool_body(h_ref, batch_ref, sums_ref, cnt_ref):
    b = batch_ref[...]
    oh = (jnp.broadcast_to(b, (_NG, _NPAD))
          == lax.broadcasted_iota(jnp.int32, (_NG, _NPAD), 0)).astype(jnp.float32)
    sums_ref[...] = jnp.dot(oh, h_ref[...], preferred_element_type=jnp.float32)
    cnt_ref[...] = jnp.dot(oh, jnp.ones((_NPAD, _HID), jnp.float32),
                           preferred_element_type=jnp.float32)


def _head_body(sums_ref, cnt_ref, meta_in_ref, graph_in_ref,
               w1m, b1m, w2m, b2m, w1g, b1g, w2g, b2g,
               f1w, f1b, f2w, f2b, f3w, f3b, out_ref):
    pooled = sums_ref[...] / jnp.maximum(cnt_ref[...], 1.0)
    meta = jnp.maximum(meta_in_ref[...] @ w1m[...] + b1m[...], 0.0)
    meta = jnp.maximum(meta @ w2m[...] + b2m[...], 0.0)
    graph = jnp.maximum(graph_in_ref[...] @ w1g[...] + b1g[...], 0.0)
    graph = jnp.maximum(graph @ w2g[...] + b2g[...], 0.0)
    z = jnp.concatenate([pooled, meta, graph], axis=1)
    z = jnp.maximum(z @ f1w[...] + f1b[...], 0.0)
    z = jnp.maximum(z @ f2w[...] + f2b[...], 0.0)
    out_ref[...] = z @ f3w[...] + f3b[...]


def _tc_call(body, out_shapes, *args):
    return pl.pallas_call(
        body,
        out_shape=out_shapes,
    )(*args)


def _edge_body(src_hbm, dst_hbm, ea_hbm, xl_hbm, xr_hbm, w_hbm,
               num_out,
               idx_s, idx_d, idx_dn, ea_v, xl_v, xr_v, den_v, w_v,
               acc_num, sem_i, sem_g, sem_sn, sem_sd):
    core = lax.axis_index("c")
    sub = lax.axis_index("s")
    wid = core * 16 + sub

    zeros16 = jnp.zeros((16,), jnp.float32)

    def zrow(j, _):
        for h in range(_H):
            den_v[j, pl.ds(h * _D, _D)] = zeros16
        return 0

    lax.fori_loop(0, _CHUNK, zrow, 0)
    row0 = sub * _APT
    for b in range(_APT // _CHUNK):
        pltpu.sync_copy(den_v, acc_num.at[pl.ds(row0 + b * _CHUNK, _CHUNK)])
    plsc.subcore_barrier()

    pltpu.sync_copy(w_hbm, w_v)
    lanes = lax.iota(jnp.int32, 16)

    def ebase(ch):
        return (wid * _CPT + ch) * _CHUNK

    def issue_idx(ch, b3):
        base = ebase(ch)
        pltpu.async_copy(src_hbm.at[pl.ds(base, _CHUNK)], idx_s.at[b3], sem_i)
        pltpu.async_copy(dst_hbm.at[pl.ds(base, _CHUNK)], idx_d.at[b3], sem_i)
        pltpu.async_copy(ea_hbm.at[pl.ds(base, _CHUNK)], ea_v.at[b3], sem_i)

    def wait_idx(ch, b3):
        base = ebase(ch)
        pltpu.make_async_copy(src_hbm.at[pl.ds(base, _CHUNK)],
                              idx_s.at[b3], sem_i).wait()
        pltpu.make_async_copy(dst_hbm.at[pl.ds(base, _CHUNK)],
                              idx_d.at[b3], sem_i).wait()
        pltpu.make_async_copy(ea_hbm.at[pl.ds(base, _CHUNK)],
                              ea_v.at[b3], sem_i).wait()

    def issue_gather(g2, b3):
        pltpu.async_copy(xl_hbm.at[idx_s.at[b3]], xl_v.at[g2], sem_g)
        pltpu.async_copy(xr_hbm.at[idx_d.at[b3]], xr_v.at[g2], sem_g)

    def wait_gather(g2, b3):
        pltpu.make_async_copy(xl_hbm.at[idx_s.at[b3]],
                              xl_v.at[g2], sem_g).wait()
        pltpu.make_async_copy(xr_hbm.at[idx_d.at[b3]],
                              xr_v.at[g2], sem_g).wait()

    def zgroup_for(b3):
        def zg(g, _):
            dst16 = idx_d[b3, pl.ds(g * 16, 16)]
            dbase = (dst16 & 7) * _D
            rows = g * 16 + lanes
            for h in range(_H):
                plsc.store_scatter(den_v, [rows, dbase + h], zeros16)
            return 0

        lax.fori_loop(0, _CHUNK // 16, zg, 0)

    def compute(g2, b3):
        xlb, xrb = xl_v.at[g2], xr_v.at[g2]

        def group(g, _):
            rows = g * 16 + lanes
            ea16 = ea_v[b3, pl.ds(g * 16, 16)]
            dst16 = idx_d[b3, pl.ds(g * 16, 16)]
            idx_dn[pl.ds(g * 16, 16)] = _NPAD + (dst16 >> 3)
            dbase = (dst16 & 7) * _D

            def hbody(h, _):
                hv = jnp.full((16,), h, jnp.int32)
                ha = jnp.full((16,), _H + h, jnp.int32)
                colbase = jnp.full((16,), h * _D, jnp.int32)
                acc = None
                xls = []
                for d in range(_D):
                    rotv = (lanes + d) & (_D - 1)
                    colv = colbase + rotv
                    xlv = plsc.load_gather(xlb, [rows, colv])
                    xrv = plsc.load_gather(xrb, [rows, colv])
                    wv = plsc.load_gather(w_v, [hv, rotv])
                    av = plsc.load_gather(w_v, [ha, rotv])
                    xls.append(xlv)
                    s = xlv + xrv + ea16 * wv
                    s = jnp.maximum(s, 0.2 * s) * av
                    acc = s if acc is None else acc + s
                exv = jnp.exp(acc)
                for d in range(_D):
                    rotv = (lanes + d) & (_D - 1)
                    plsc.store_scatter(xrb, [rows, colbase + rotv],
                                       exv * xls[d])
                plsc.store_scatter(den_v, [rows, dbase + h], exv)
                return 0

            lax.fori_loop(0, _H, hbody, 0)
            return 0

        lax.fori_loop(0, _CHUNK // 16, group, 0)

    issue_idx(0, 0)
    wait_idx(0, 0)
    issue_gather(0, 0)
    issue_idx(1, 1)

    def chunk_six(c6, _):
        for par in range(6):
            ch = c6 * 6 + par
            g2, b3 = par % 2, par % 3
            wait_gather(g2, b3)

            @pl.when(ch + 1 < _CPT)
            def _():
                wait_idx(ch + 1, (par + 1) % 3)
                issue_gather(1 - g2, (par + 1) % 3)

            @pl.when(ch + 2 < _CPT)
            def _():
                issue_idx(ch + 2, (par + 2) % 3)

            compute(g2, b3)
            pltpu.sync_copy(den_v, acc_num.at[idx_dn], add=True)
            zgroup_for(b3)
            pltpu.sync_copy(xr_v.at[g2], acc_num.at[idx_d.at[b3]], add=True)
        return 0

    lax.fori_loop(0, _CPT // 6, chunk_six, 0)
    plsc.subcore_barrier()

    for b in range(_APT // _CHUNK):
        r0 = row0 + b * _CHUNK
        pltpu.sync_copy(acc_num.at[pl.ds(r0, _CHUNK)], xl_v.at[0])
        pltpu.sync_copy(xl_v.at[0], num_out.at[core, pl.ds(r0, _CHUNK)])


@functools.cache
def _edge_pass():
  return pl.kernel(
    _edge_body,
    out_type=jax.ShapeDtypeStruct((2, _NTOT, _HID), jnp.float32),
    mesh=plsc.VectorSubcoreMesh(core_axis_name="c", subcore_axis_name="s"),
    compiler_params=pltpu.CompilerParams(needs_layout_passes=False),
    scratch_types=[
        pltpu.VMEM((3, _CHUNK), jnp.int32),
        pltpu.VMEM((3, _CHUNK), jnp.int32),
        pltpu.VMEM((_CHUNK,), jnp.int32),
        pltpu.VMEM((3, _CHUNK), jnp.float32),
        pltpu.VMEM((2, _CHUNK, _HID), jnp.float32),
        pltpu.VMEM((2, _CHUNK, _HID), jnp.float32),
        pltpu.VMEM((_CHUNK, _HID), jnp.float32),
        pltpu.VMEM((16, 16), jnp.float32),
        pltpu.VMEM_SHARED((_NTOT, _HID), jnp.float32),
        pltpu.SemaphoreType.DMA,
        pltpu.SemaphoreType.DMA,
        pltpu.SemaphoreType.DMA,
        pltpu.SemaphoreType.DMA,
    ],
  )


def _layer(h, src_p, dst_p, ea_p, xs):
    Wl, bl, Wr, br, ew, att, wtab, bias, g, b, alpha = xs
    xl, xr, c = _tc_call(
        _prep_body,
        [jax.ShapeDtypeStruct((_NPAD, _HID), jnp.float32),
         jax.ShapeDtypeStruct((_NPAD, _HID), jnp.float32),
         jax.ShapeDtypeStruct((_NPAD, _HID), jnp.float32)],
        h, Wl, bl, Wr, br, ew, att)
    acc2 = _edge_pass()(src_p, dst_p, ea_p, xl, xr, wtab)
    num2 = acc2[:, :_NPAD, :]
    den2 = acc2[:, _NPAD:, :].reshape(2, _NPAD, _D)
    out, s1, s2 = _tc_call(
        _combine_body,
        [jax.ShapeDtypeStruct((_NPAD, _HID), jnp.float32),
         jax.ShapeDtypeStruct((1, _HID), jnp.float32),
         jax.ShapeDtypeStruct((1, _HID), jnp.float32)],
        num2, den2, xl, c, bias)
    return _tc_call(
        _norm_body, jax.ShapeDtypeStruct((_NPAD, _HID), jnp.float32),
        out, s1, s2, g, b, alpha, h)


def kernel(x, edge_index, edge_attr, batch, global_features, params):
    ea_mean = edge_attr.mean()

    x_p = jnp.zeros((_NPAD, 8), jnp.float32).at[:_N, :4].set(x)
    pad_e = _EPAD - _E
    src_p = jnp.concatenate([edge_index[0], jnp.zeros((pad_e,), jnp.int32)])
    dst_p = jnp.concatenate(
        [edge_index[1], jnp.full((pad_e,), _NPAD - 1, jnp.int32)])
    ea_p = jnp.concatenate([edge_attr[:, 0], jnp.zeros((pad_e,), jnp.float32)])
    batch_p = jnp.concatenate(
        [batch, jnp.full((_NPAD - _N,), -1, jnp.int32)]).reshape(1, _NPAD)

    Wls, bls, Wrs, brs, ews, atts, wtabs, biases, gs, bs = (
        [] for _ in range(10))
    for i, name in enumerate(("gat1", "gat2", "gat3", "gat4")):
        p = params[name]
        Wl, Wr = p["Wl"], p["Wr"]
        if i == 0:
            Wl = jnp.zeros((_HID, _HID), jnp.float32).at[:64].set(Wl)
            Wr = jnp.zeros((_HID, _HID), jnp.float32).at[:64].set(Wr)
        Wls.append(Wl)
        Wrs.append(Wr)
        bls.append(p["bl"].reshape(1, _HID))
        brs.append(p["br"].reshape(1, _HID))
        ews.append((ea_mean * p["We"][0]).reshape(1, _HID))
        atts.append(p["att"].reshape(1, _HID))
        wtabs.append(jnp.concatenate([p["We"].reshape(_H, _D), p["att"]], 0))
        biases.append(p["bias"].reshape(1, _HID))
        g, b = params["bn" + str(i + 1)]
        gs.append(g.reshape(1, _HID))
        bs.append(b.reshape(1, _HID))
    xs = tuple(jnp.stack(v) for v in
               (Wls, bls, Wrs, brs, ews, atts, wtabs, biases, gs, bs))
    xs = xs + (jnp.array([0.0, 1.0, 1.0, 1.0],
                         jnp.float32).reshape(4, 1, 1),)

    We_, be_ = params["embed"]
    We_p = jnp.zeros((8, 64), jnp.float32).at[:4].set(We_)

    h = _tc_call(_embed_body,
                 jax.ShapeDtypeStruct((_NPAD, _HID), jnp.float32),
                 x_p, We_p, be_.reshape(1, -1))

    def body(hc, x):
        return _layer(hc, src_p, dst_p, ea_p, x), None

    h, _ = lax.scan(body, h, xs)

    sums, cnt = _tc_call(
        _pool_body,
        [jax.ShapeDtypeStruct((_NG, _HID), jnp.float32),
         jax.ShapeDtypeStruct((_NG, _HID), jnp.float32)],
        h, batch_p)

    gf = global_features.squeeze(1)
    p = params
    return _tc_call(
        _head_body, jax.ShapeDtypeStruct((_NG, 1), jnp.float32),
        sums, cnt, gf[:, 0:4], gf[:, 4:6],
        p["meta1"][0], p["meta1"][1], p["meta2"][0], p["meta2"][1],
        p["graph1"][0], p["graph1"][1], p["graph2"][0], p["graph2"][1],
        p["fc1"][0], p["fc1"][1], p["fc2"][0], p["fc2"][1],
        p["fc3"][0], p["fc3"][1])
